# baseline (speedup 1.0000x reference)
import functools

import jax
import jax.numpy as jnp
from jax import lax
from jax.experimental import pallas as pl
from jax.experimental.pallas import tpu as pltpu
import jax.experimental.pallas.tpu_sc as plsc

MAX_REL = 2048
Q_LEN = 32
K_LEN = 2048
D_MODEL = 1024

NC, NS = 2, 16
NW = NC * NS
LANES = 16

CHUNK = 8
NCHUNK = K_LEN // CHUNK
NBUF = 8


def _sc_body(rel_hbm, out_hbm, idx_v, rows_v, *sems):
    gsems, ssems = sems[:NBUF], sems[NBUF:]
    w = lax.axis_index("s") * NC + lax.axis_index("c")
    base = MAX_REL + w

    def build_idx(v, carry):
        start = jnp.full((LANES,), base, jnp.int32) - v * LANES
        idx_v[pl.ds(v * LANES, LANES)] = start - lax.iota(jnp.int32, LANES)
        return carry

    lax.fori_loop(0, K_LEN // LANES, build_idx, 0)

    def start_gather(c, b):
        idx_slice = idx_v.at[pl.ds(c * CHUNK, CHUNK)]
        pltpu.async_copy(rel_hbm.at[idx_slice], rows_v.at[b], gsems[b])

    def wait_gather(b):
        pltpu.make_async_copy(
            rel_hbm.at[idx_v.at[pl.ds(0, CHUNK)]], rows_v.at[b], gsems[b]
        ).wait()

    def start_store(c, b):
        pltpu.async_copy(rows_v.at[b], out_hbm.at[w, pl.ds(c * CHUNK, CHUNK)],
                         ssems[b])

    def wait_store(b):
        pltpu.make_async_copy(
            rows_v.at[b], out_hbm.at[w, pl.ds(0, CHUNK)], ssems[b]
        ).wait()

    for b in range(NBUF):
        start_gather(b, b)

    def ring(h, carry):
        c0 = h * NBUF
        for b in range(NBUF):
            wait_gather(b)
            start_store(c0 + b, b)
        for b in range(NBUF):
            wait_store(b)

            @pl.when(c0 + b + NBUF < NCHUNK)
            def _():
                start_gather(c0 + b + NBUF, b)

        return carry

    lax.fori_loop(0, NCHUNK // NBUF, ring, 0)


@functools.partial(jax.jit, static_argnames=())
def _sc_gather(rel_emb):
    mesh = plsc.VectorSubcoreMesh(core_axis_name="c", subcore_axis_name="s")
    run = pl.kernel(
        _sc_body,
        out_type=jax.ShapeDtypeStruct((Q_LEN, K_LEN, D_MODEL), jnp.float32),
        mesh=mesh,
        scratch_types=(
            [pltpu.VMEM((K_LEN,), jnp.int32),
             pltpu.VMEM((NBUF, CHUNK, D_MODEL), jnp.float32)]
            + [pltpu.SemaphoreType.DMA] * (2 * NBUF)
        ),
    )
    return run(rel_emb)


def kernel(q, k, rel_emb):
    del q, k
    return _sc_gather(rel_emb)


# speedup vs baseline: 1.0095x; 1.0095x over previous
import functools

import jax
import jax.numpy as jnp
from jax import lax
from jax.experimental import pallas as pl
from jax.experimental.pallas import tpu as pltpu
import jax.experimental.pallas.tpu_sc as plsc

MAX_REL = 2048
Q_LEN = 32
K_LEN = 2048
D_MODEL = 1024

NC, NS = 2, 16
NW = NC * NS
LANES = 16

CHUNK = 16
NCHUNK = K_LEN // CHUNK
NBUF = 4


def _sc_body(rel_hbm, out_hbm, idx_v, pidx_v, rows_v, psem, *sems):
    gsems, ssems = sems[:NBUF], sems[NBUF:]
    w = lax.axis_index("s") * NC + lax.axis_index("c")
    wp1 = w + 1
    pad = lax.rem(8 - lax.rem(wp1, 8), 8)
    a1 = pl.multiple_of(wp1 + pad, 8)

    def build_idx(c, carry):
        j = jnp.full((LANES,), K_LEN - 1 - c * CHUNK, jnp.int32) - pad
        j = j - lax.iota(jnp.int32, LANES)
        idx_v[c, pl.ds(0, LANES)] = jnp.where(j < 0, K_LEN - 1, j)
        return carry

    lax.fori_loop(0, NCHUNK, build_idx, 0)

    pidx_v[pl.ds(0, LANES)] = jnp.full((LANES,), w + 8, jnp.int32) - lax.iota(
        jnp.int32, LANES)
    prefix_gather = pltpu.async_copy(
        rel_hbm.at[pidx_v.at[pl.ds(0, 8)]], rows_v.at[NBUF, pl.ds(0, 8)], psem)

    def start_read(c, b):
        src = rel_hbm.at[pl.ds(a1 + c * CHUNK, CHUNK)]
        pltpu.async_copy(src, rows_v.at[b], gsems[b])

    def wait_read(b):
        pltpu.make_async_copy(
            rel_hbm.at[pl.ds(0, CHUNK)], rows_v.at[b], gsems[b]
        ).wait()

    def start_scatter(c, b):
        pltpu.async_copy(rows_v.at[b], out_hbm.at[w].at[idx_v.at[c]],
                         ssems[b])

    def wait_scatter(b):
        pltpu.make_async_copy(
            rows_v.at[b], out_hbm.at[w].at[idx_v.at[0]], ssems[b]
        ).wait()

    for b in range(NBUF):
        start_read(b, b)

    def ring(h, carry):
        c0 = h * NBUF
        for b in range(NBUF):
            wait_read(b)
            start_scatter(c0 + b, b)
        for b in range(NBUF):
            wait_scatter(b)

            @pl.when(c0 + b + NBUF < NCHUNK)
            def _():
                start_read(c0 + b + NBUF, b)

        return carry

    lax.fori_loop(0, NCHUNK // NBUF, ring, 0)

    prefix_gather.wait()
    pltpu.sync_copy(rows_v.at[NBUF, pl.ds(0, 8)],
                    out_hbm.at[w, pl.ds(K_LEN - 8, 8)])


@functools.partial(jax.jit, static_argnames=())
def _sc_gather(rel_emb):
    mesh = plsc.VectorSubcoreMesh(core_axis_name="c", subcore_axis_name="s")
    run = pl.kernel(
        _sc_body,
        out_type=jax.ShapeDtypeStruct((Q_LEN, K_LEN, D_MODEL), jnp.float32),
        mesh=mesh,
        scratch_types=(
            [pltpu.VMEM((NCHUNK, CHUNK), jnp.int32),
             pltpu.VMEM((LANES,), jnp.int32),
             pltpu.VMEM((NBUF + 1, CHUNK, D_MODEL), jnp.float32),
             pltpu.SemaphoreType.DMA]
            + [pltpu.SemaphoreType.DMA] * (2 * NBUF)
        ),
    )
    return run(rel_emb)


def kernel(q, k, rel_emb):
    del q, k
    return _sc_gather(rel_emb)
